# baseline (speedup 1.0000x reference)
import jax
import jax.numpy as jnp
import numpy as np
from jax.experimental import pallas as pl
from jax.experimental.pallas import tpu as pltpu

_TOTAL_COMPONENTS = 128
_TOTAL_NODES = 50000
_TOTAL_EDGES = 800000

_GRID = 2
_HFB = 10000
_ELB = 320000
_ZR = 5000
_EFL = 80000


def kernel(node_features, edge_index, node_sizes, edge_sizes):
    num_nodes, d = node_features.shape
    num_edges = edge_index.shape[1]
    num_components = node_sizes.shape[0]
    pad_nodes = _TOTAL_NODES - num_nodes
    pad_edges = _TOTAL_EDGES - num_edges
    tail = _TOTAL_COMPONENTS - num_components - 1
    half_rows = num_nodes // 2

    def body(nfa_ref, nfb_ref, ei_ref, ns_ref, es_ref,
             pf_ref, pei_ref, pns_ref, pes_ref,
             zfill, efill, sems, fill_sems):
        i = pl.program_id(0)

        def fill_copies():
            return [
                pltpu.make_async_copy(
                    zfill, pf_ref.at[pl.ds(num_nodes + k * _ZR, _ZR)],
                    fill_sems.at[k])
                for k in range(2)
            ] + [
                pltpu.make_async_copy(
                    efill,
                    pei_ref.at[:, pl.ds(num_edges + k * _EFL, _EFL)],
                    fill_sems.at[2 + k])
                for k in range(2)
            ]

        @pl.when(i == 0)
        def _():
            zfill[...] = jnp.zeros_like(zfill)
            efill[...] = jnp.full_like(efill, num_nodes)
            for c in fill_copies():
                c.start()
            idt = node_sizes.dtype
            pns_ref[...] = jnp.concatenate([
                ns_ref[...], jnp.full((1,), pad_nodes, idt),
                jnp.zeros((tail,), idt)])
            pes_ref[...] = jnp.concatenate([
                es_ref[...], jnp.full((1,), pad_edges, idt),
                jnp.zeros((tail,), idt)])

        ca = pltpu.make_async_copy(
            nfa_ref, pf_ref.at[pl.ds(i * _HFB, _HFB)], sems.at[0])
        cb = pltpu.make_async_copy(
            nfb_ref, pf_ref.at[pl.ds(half_rows + i * _HFB, _HFB)],
            sems.at[1])
        ce = pltpu.make_async_copy(
            ei_ref, pei_ref.at[:, pl.ds(i * _ELB, _ELB)], sems.at[2])
        ca.start()
        cb.start()
        ce.start()
        ca.wait()
        cb.wait()
        ce.wait()

        @pl.when(i == _GRID - 1)
        def _():
            for c in fill_copies():
                c.wait()

    out = pl.pallas_call(
        body,
        grid=(_GRID,),
        out_shape=[
            jax.ShapeDtypeStruct((_TOTAL_NODES, d), node_features.dtype),
            jax.ShapeDtypeStruct((2, _TOTAL_EDGES), edge_index.dtype),
            jax.ShapeDtypeStruct((_TOTAL_COMPONENTS,), node_sizes.dtype),
            jax.ShapeDtypeStruct((_TOTAL_COMPONENTS,), edge_sizes.dtype),
        ],
        in_specs=[
            pl.BlockSpec((_HFB, d), lambda i: (i, 0)),
            pl.BlockSpec((_HFB, d), lambda i: (i + 2, 0)),
            pl.BlockSpec((2, _ELB), lambda i: (0, i)),
            pl.BlockSpec((num_components,), lambda i: (0,)),
            pl.BlockSpec((num_components,), lambda i: (0,)),
        ],
        out_specs=[
            pl.BlockSpec(memory_space=pl.ANY),
            pl.BlockSpec(memory_space=pl.ANY),
            pl.BlockSpec((_TOTAL_COMPONENTS,), lambda i: (0,)),
            pl.BlockSpec((_TOTAL_COMPONENTS,), lambda i: (0,)),
        ],
        scratch_shapes=[
            pltpu.VMEM((_ZR, d), node_features.dtype),
            pltpu.VMEM((2, _EFL), edge_index.dtype),
            pltpu.SemaphoreType.DMA((3,)),
            pltpu.SemaphoreType.DMA((4,)),
        ],
    )(node_features, node_features, edge_index, node_sizes, edge_sizes)

    padded_features, padded_edge_index, padded_node_sizes, \
        padded_edge_sizes = out

    component_mask = jnp.asarray(
        np.arange(_TOTAL_COMPONENTS) < num_components)

    return (
        padded_features,
        padded_edge_index,
        padded_node_sizes,
        padded_edge_sizes,
        component_mask,
    )


# speedup vs baseline: 1.0309x; 1.0309x over previous
import jax
import jax.numpy as jnp
import numpy as np
from jax.experimental import pallas as pl
from jax.experimental.pallas import tpu as pltpu

_TOTAL_COMPONENTS = 128
_TOTAL_NODES = 50000
_TOTAL_EDGES = 800000

_GRID = 3
_FB = 20000
_ELB = 320000
_COPY_BLOCKS = 2


def kernel(node_features, edge_index, node_sizes, edge_sizes):
    num_nodes, d = node_features.shape
    num_edges = edge_index.shape[1]
    num_components = node_sizes.shape[0]
    pad_nodes = _TOTAL_NODES - num_nodes
    pad_edges = _TOTAL_EDGES - num_edges

    tail = _TOTAL_COMPONENTS - num_components - 1

    def body(nf_ref, ei_ref, ns_ref, es_ref,
             pf_ref, pei_ref, pns_ref, pes_ref):
        i = pl.program_id(0)

        @pl.when(i < _COPY_BLOCKS)
        def _():
            pf_ref[...] = nf_ref[...]
            pei_ref[...] = ei_ref[...]

        @pl.when(i >= _COPY_BLOCKS)
        def _():
            pf_ref[...] = jnp.zeros_like(pf_ref)
            pei_ref[...] = jnp.full_like(pei_ref, num_nodes)

        @pl.when(i == 0)
        def _():
            idt = node_sizes.dtype
            pns_ref[...] = jnp.concatenate([
                ns_ref[...], jnp.full((1,), pad_nodes, idt),
                jnp.zeros((tail,), idt)])
            pes_ref[...] = jnp.concatenate([
                es_ref[...], jnp.full((1,), pad_edges, idt),
                jnp.zeros((tail,), idt)])

    clamp = _COPY_BLOCKS - 1

    padded_features, padded_edge_index, padded_node_sizes, \
        padded_edge_sizes = pl.pallas_call(
            body,
            grid=(_GRID,),
            out_shape=[
                jax.ShapeDtypeStruct((_TOTAL_NODES, d),
                                     node_features.dtype),
                jax.ShapeDtypeStruct((2, _TOTAL_EDGES), edge_index.dtype),
                jax.ShapeDtypeStruct((_TOTAL_COMPONENTS,),
                                     node_sizes.dtype),
                jax.ShapeDtypeStruct((_TOTAL_COMPONENTS,),
                                     edge_sizes.dtype),
            ],
            in_specs=[
                pl.BlockSpec((_FB, d),
                             lambda i: (jnp.minimum(i, clamp), 0)),
                pl.BlockSpec((2, _ELB),
                             lambda i: (0, jnp.minimum(i, clamp))),
                pl.BlockSpec((num_components,), lambda i: (0,)),
                pl.BlockSpec((num_components,), lambda i: (0,)),
            ],
            out_specs=[
                pl.BlockSpec((_FB, d), lambda i: (i, 0)),
                pl.BlockSpec((2, _ELB), lambda i: (0, i)),
                pl.BlockSpec((_TOTAL_COMPONENTS,), lambda i: (0,)),
                pl.BlockSpec((_TOTAL_COMPONENTS,), lambda i: (0,)),
            ],
        )(node_features, edge_index, node_sizes, edge_sizes)

    component_mask = jnp.asarray(
        np.arange(_TOTAL_COMPONENTS) < num_components)

    return (
        padded_features,
        padded_edge_index,
        padded_node_sizes,
        padded_edge_sizes,
        component_mask,
    )


# speedup vs baseline: 1.0337x; 1.0027x over previous
import jax
import jax.numpy as jnp
import numpy as np
from jax.experimental import pallas as pl
from jax.experimental.pallas import tpu as pltpu

_TOTAL_COMPONENTS = 128
_TOTAL_NODES = 50000
_TOTAL_EDGES = 800000

_GRID = 3
_FB = 20000
_ELB = 320000
_COPY_BLOCKS = 2


def kernel(node_features, edge_index, node_sizes, edge_sizes):
    num_nodes, d = node_features.shape
    num_edges = edge_index.shape[1]
    num_components = node_sizes.shape[0]
    pad_nodes = _TOTAL_NODES - num_nodes
    pad_edges = _TOTAL_EDGES - num_edges

    tail = _TOTAL_COMPONENTS - num_components - 1

    def body(nf_ref, ei_ref, ns_ref, es_ref,
             pf_ref, pei_ref, pns_ref, pes_ref):
        i = pl.program_id(0)
        is_copy = i < _COPY_BLOCKS
        pf_ref[...] = jnp.where(is_copy, nf_ref[...], 0.0)
        pei_ref[...] = jnp.where(is_copy, ei_ref[...], num_nodes)

        @pl.when(i == 0)
        def _():
            idt = node_sizes.dtype
            pns_ref[...] = jnp.concatenate([
                ns_ref[...], jnp.full((1,), pad_nodes, idt),
                jnp.zeros((tail,), idt)])
            pes_ref[...] = jnp.concatenate([
                es_ref[...], jnp.full((1,), pad_edges, idt),
                jnp.zeros((tail,), idt)])

    clamp = _COPY_BLOCKS - 1

    padded_features, padded_edge_index, padded_node_sizes, \
        padded_edge_sizes = pl.pallas_call(
            body,
            grid=(_GRID,),
            out_shape=[
                jax.ShapeDtypeStruct((_TOTAL_NODES, d),
                                     node_features.dtype),
                jax.ShapeDtypeStruct((2, _TOTAL_EDGES), edge_index.dtype),
                jax.ShapeDtypeStruct((_TOTAL_COMPONENTS,),
                                     node_sizes.dtype),
                jax.ShapeDtypeStruct((_TOTAL_COMPONENTS,),
                                     edge_sizes.dtype),
            ],
            in_specs=[
                pl.BlockSpec((_FB, d),
                             lambda i: (jnp.minimum(i, clamp), 0)),
                pl.BlockSpec((2, _ELB),
                             lambda i: (0, jnp.minimum(i, clamp))),
                pl.BlockSpec((num_components,), lambda i: (0,)),
                pl.BlockSpec((num_components,), lambda i: (0,)),
            ],
            out_specs=[
                pl.BlockSpec((_FB, d), lambda i: (i, 0)),
                pl.BlockSpec((2, _ELB), lambda i: (0, i)),
                pl.BlockSpec((_TOTAL_COMPONENTS,), lambda i: (0,)),
                pl.BlockSpec((_TOTAL_COMPONENTS,), lambda i: (0,)),
            ],
        )(node_features, edge_index, node_sizes, edge_sizes)

    component_mask = jnp.asarray(
        np.arange(_TOTAL_COMPONENTS) < num_components)

    return (
        padded_features,
        padded_edge_index,
        padded_node_sizes,
        padded_edge_sizes,
        component_mask,
    )
